# baseline (speedup 1.0000x reference)
import jax
import jax.numpy as jnp
from jax.experimental import pallas as pl
from jax.experimental.pallas import tpu as pltpu

N = 10000
D = 256
BM = 400
NB = N // BM
CW = 2560
NC = 4
NP = CW * NC
PADC = NP - N


def _fused_kernel(a0_ref, a1_ref, a2_ref, a3_ref, x0_ref, e1_ref, e2_ref,
                  osum_ref, e1_s):
    i = pl.program_id(0)
    a_refs = (a0_ref, a1_ref, a2_ref, a3_ref)

    if NP > N:
        @pl.when(i == 0)
        def _():
            e1_s[pl.ds(N, NP - N), :] = jnp.zeros((NP - N, D), jnp.float32)

        @pl.when(i < 4)
        def _():
            col = jax.lax.broadcasted_iota(jnp.int32, (BM, CW), 1)
            a3_ref[...] = jnp.where(col < CW - PADC, a3_ref[...], 0.0)

    @pl.when(i < NB)
    def _():
        acc = jnp.zeros((BM, D), jnp.float32)
        for c in range(NC):
            acc += jnp.dot(a_refs[c][...], x0_ref[pl.ds(c * CW, CW), :],
                           preferred_element_type=jnp.float32)
        e1_ref[...] = acc
        e1_s[pl.ds(i * BM, BM), :] = acc

    @pl.when(i >= NB)
    def _():
        j = i - NB
        acc = jnp.zeros((BM, D), jnp.float32)
        for c in range(NC):
            acc += jnp.dot(a_refs[c][...], e1_s[pl.ds(c * CW, CW), :],
                           preferred_element_type=jnp.float32)
        e2_ref[...] = acc
        osum_ref[...] = (
            x0_ref[pl.ds(j * BM, BM), :] + e1_s[pl.ds(j * BM, BM), :] + acc)


def kernel(encoder_adj, init_emb):
    x0p = jnp.pad(init_emb, ((0, NP - N), (0, 0)))

    a_specs = [
        pl.BlockSpec((BM, CW), lambda i, c=c: (i % NB, c)) for c in range(NC)
    ]
    x0_spec = pl.BlockSpec((NP, D), lambda i: (0, 0))
    e1_spec = pl.BlockSpec((BM, D), lambda i: (jnp.minimum(i, NB - 1), 0))
    out2_spec = pl.BlockSpec((BM, D), lambda i: (jnp.maximum(i - NB, 0), 0))

    e1, e2, summed = pl.pallas_call(
        _fused_kernel,
        grid=(2 * NB,),
        in_specs=a_specs + [x0_spec],
        out_specs=[e1_spec, out2_spec, out2_spec],
        out_shape=[
            jax.ShapeDtypeStruct((N, D), jnp.float32),
            jax.ShapeDtypeStruct((N, D), jnp.float32),
            jax.ShapeDtypeStruct((N, D), jnp.float32),
        ],
        scratch_shapes=[pltpu.VMEM((NP, D), jnp.float32)],
    )(*([encoder_adj] * NC), x0p)

    return (summed, init_emb, e1, e2)


# speedup vs baseline: 1.0283x; 1.0283x over previous
import jax
import jax.numpy as jnp
from jax.experimental import pallas as pl
from jax.experimental.pallas import tpu as pltpu

N = 10000
D = 256
BM = 400
NB = N // BM


def _fused_kernel(a_ref, x0_full_ref, e1_ref, e2_ref,
                  osum_ref, e1_scratch):
    i = pl.program_id(0)

    @pl.when(i < NB)
    def _():
        e1_blk = jnp.dot(a_ref[...], x0_full_ref[...],
                         preferred_element_type=jnp.float32)
        e1_ref[...] = e1_blk
        e1_scratch[pl.ds(i * BM, BM), :] = e1_blk

    @pl.when(i >= NB)
    def _():
        j = i - NB
        e2_blk = jnp.dot(a_ref[...], e1_scratch[...],
                         preferred_element_type=jnp.float32)
        e2_ref[...] = e2_blk
        osum_ref[...] = (
            x0_full_ref[pl.ds(j * BM, BM), :]
            + e1_scratch[pl.ds(j * BM, BM), :] + e2_blk)


def kernel(encoder_adj, init_emb):
    a_spec = pl.BlockSpec((BM, N), lambda i: (i % NB, 0))
    x0_full_spec = pl.BlockSpec((N, D), lambda i: (0, 0))
    e1_spec = pl.BlockSpec((BM, D), lambda i: (jnp.minimum(i, NB - 1), 0))
    out2_spec = pl.BlockSpec((BM, D), lambda i: (jnp.maximum(i - NB, 0), 0))

    e1, e2, summed = pl.pallas_call(
        _fused_kernel,
        grid=(2 * NB,),
        in_specs=[a_spec, x0_full_spec],
        out_specs=[e1_spec, out2_spec, out2_spec],
        out_shape=[
            jax.ShapeDtypeStruct((N, D), jnp.float32),
            jax.ShapeDtypeStruct((N, D), jnp.float32),
            jax.ShapeDtypeStruct((N, D), jnp.float32),
        ],
        scratch_shapes=[pltpu.VMEM((N, D), jnp.float32)],
    )(encoder_adj, init_emb)

    return (summed, init_emb, e1, e2)
